# baseline (speedup 1.0000x reference)
import functools

import jax
import jax.numpy as jnp
from jax import lax
from jax.experimental import pallas as pl
from jax.experimental.pallas import tpu as pltpu
from jax.experimental.pallas import tpu_sc as plsc

B = 256
I = 128
F = 256
N_PROT = 256
L_PROT = 512
D = 256
M = 16384
INC_PAD = -510

BI = B * I
NC = 2
NS = 16
CHUNK = BI // NS
BLK = 128
NBLK = CHUNK // BLK
HALF = D // 2
ROWS_PER_SC = M // NC
ROWS_PER_TILE = ROWS_PER_SC // NS
CPAD = CHUNK + 2 * BLK
GB = 32


def _gate_body(b1_ref, pe_ref, w_ref, pidx_ref, ip_ref, gl_ref,
               gate_ref, prow_ref):
    x = pe_ref[...]
    w = w_ref[0]
    g = jnp.sum(x * w, axis=-1) + b1_ref[0]
    g = jnp.maximum(g, 0.0)
    m = (gl_ref[...] != INC_PAD).astype(jnp.float32)
    gate_ref[...] = g * m
    prow_ref[...] = pidx_ref[...] * L_PROT + ip_ref[...]


def _gate_call(pe, wt, b1, pidx, ip, gl):
    return pl.pallas_call(
        _gate_body,
        grid=(B // GB,),
        in_specs=[
            pl.BlockSpec(memory_space=pltpu.SMEM),
            pl.BlockSpec((GB, I, F), lambda i: (i, 0, 0)),
            pl.BlockSpec((1, F), lambda i: (0, 0)),
            pl.BlockSpec((GB, 1), lambda i: (i, 0)),
            pl.BlockSpec((GB, I), lambda i: (i, 0)),
            pl.BlockSpec((GB, I), lambda i: (i, 0)),
        ],
        out_specs=[
            pl.BlockSpec((GB, I), lambda i: (i, 0)),
            pl.BlockSpec((GB, I), lambda i: (i, 0)),
        ],
        out_shape=[
            jax.ShapeDtypeStruct((B, I), jnp.float32),
            jax.ShapeDtypeStruct((B, I), jnp.int32),
        ],
    )(b1, pe, wt, pidx, ip, gl)


_mesh = plsc.VectorSubcoreMesh(core_axis_name="c", subcore_axis_name="s")


@functools.partial(
    pl.kernel,
    out_type=jax.ShapeDtypeStruct((M, D), jnp.float32),
    mesh=_mesh,
    scratch_types=[
        pltpu.VMEM_SHARED((ROWS_PER_SC, HALF), jnp.float32),
        pltpu.VMEM((CHUNK,), jnp.int32),
        pltpu.VMEM((CHUNK,), jnp.int32),
        pltpu.VMEM((CHUNK,), jnp.float32),
        pltpu.VMEM((CPAD,), jnp.int32),
        pltpu.VMEM((CPAD,), jnp.int32),
        pltpu.VMEM((CPAD,), jnp.float32),
        pltpu.VMEM((BLK,), jnp.int32),
        pltpu.VMEM((BLK,), jnp.int32),
        pltpu.VMEM((BLK,), jnp.int32),
        pltpu.VMEM((BLK,), jnp.int32),
        pltpu.VMEM((BLK, HALF), jnp.float32),
        pltpu.VMEM((BLK, HALF), jnp.float32),
        pltpu.VMEM((BLK, HALF), jnp.float32),
        pltpu.SemaphoreType.DMA,
        pltpu.SemaphoreType.DMA,
        pltpu.SemaphoreType.DMA,
        pltpu.SemaphoreType.DMA,
    ],
    compiler_params=pltpu.CompilerParams(needs_layout_passes=False),
)
def _sc_scatter(hprot_hbm, glig_hbm, prow_hbm, gate_hbm, out_hbm,
                acc, glig_v, prow_v, gate_v, cprow, csidx, cgate,
                gidx_a, sidx_a, gidx_b2, sidx_b2,
                rows_a, rows_b, zeros_v, sem_a, sem_b, sem_sa, sem_sb):
    c = lax.axis_index("c")
    s = lax.axis_index("s")
    lo = c * ROWS_PER_SC
    base = s * CHUNK

    pltpu.sync_copy(glig_hbm.at[pl.ds(base, CHUNK)], glig_v)
    pltpu.sync_copy(prow_hbm.at[pl.ds(base, CHUNK)], prow_v)
    pltpu.sync_copy(gate_hbm.at[pl.ds(base, CHUNK)], gate_v)

    zero16 = jnp.zeros((16,), jnp.float32)
    lane = [jnp.full((16,), rr, jnp.int32) for rr in range(16)]

    def _zrow(r, carry):
        for k in range(HALF // 16):
            zeros_v[r, pl.ds(k * 16, 16)] = zero16
        return carry
    lax.fori_loop(0, BLK, _zrow, 0)

    iota16 = lax.iota(jnp.int32, 16)
    izero16 = jnp.zeros((16,), jnp.int32)
    hi = lo + ROWS_PER_SC

    def _compact(g, off):
        o = g * 16
        glv = glig_v[pl.ds(o, 16)]
        owned = jnp.logical_and(glv >= lo, glv < hi)
        _, perm = plsc.sort_key_val(owned.astype(jnp.int32), iota16,
                                    descending=True)
        pr16 = prow_v[pl.ds(o, 16)].at[perm].get(mode="promise_in_bounds")
        si16 = (glv - lo).at[perm].get(mode="promise_in_bounds")
        ga16 = gate_v[pl.ds(o, 16)].at[perm].get(mode="promise_in_bounds")
        sl = pl.ds(off, 16)
        cprow[sl] = pr16
        csidx[sl] = si16
        cgate[sl] = ga16
        return off + plsc.all_reduce_population_count(owned)[0]
    n = lax.fori_loop(0, CHUNK // 16, _compact, 0)

    def _pad(k, carry):
        sl = pl.ds(n + k * 16, 16)
        cprow[sl] = izero16
        csidx[sl] = izero16
        cgate[sl] = zero16
        return carry
    lax.fori_loop(0, (BLK // 16) + 1, _pad, 0)

    vnblk = (n + BLK - 1) // BLK

    def _build(jb, gidx_x, sidx_x):
        b0 = jb * BLK

        def _g16(k, c2):
            o = k * 16
            gidx_x[pl.ds(o, 16)] = cprow[pl.ds(b0 + o, 16)]
            sidx_x[pl.ds(o, 16)] = csidx[pl.ds(b0 + o, 16)]
            return c2
        lax.fori_loop(0, BLK // 16, _g16, 0)

    def _scale(jb, rows_x):
        b0 = jb * BLK

        def _row(t, c2):
            mg16 = cgate[pl.ds(b0 + t * 16, 16)]
            for rr in range(16):
                gv = mg16.at[lane[rr]].get(mode="promise_in_bounds")
                r = t * 16 + rr
                for k in range(HALF // 16):
                    sl = pl.ds(k * 16, 16)
                    rows_x[r, sl] = rows_x[r, sl] * gv
            return c2
        lax.fori_loop(0, BLK // 16, _row, 0)

    for p in range(2):
        src = lambda gidx_x: hprot_hbm.at[gidx_x, pl.ds(p * HALF, HALF)]

        for q in range(ROWS_PER_TILE // BLK):
            pltpu.sync_copy(
                zeros_v, acc.at[pl.ds(s * ROWS_PER_TILE + q * BLK, BLK)])
        plsc.subcore_barrier()

        @pl.when(vnblk > 0)
        def _():
            _build(0, gidx_a, sidx_a)
            pltpu.async_copy(src(gidx_a), rows_a, sem_a)

        def _pair(j, carry):
            e = j * 2

            @pl.when(jnp.logical_and(j > 0, e - 1 < vnblk))
            def _():
                pltpu.make_async_copy(rows_b, acc.at[sidx_b2],
                                      sem_sb).wait()

            @pl.when(e + 1 < vnblk)
            def _():
                _build(e + 1, gidx_b2, sidx_b2)
                pltpu.async_copy(src(gidx_b2), rows_b, sem_b)

            @pl.when(e < vnblk)
            def _():
                pltpu.make_async_copy(src(gidx_a), rows_a, sem_a).wait()
                _scale(e, rows_a)
                pltpu.async_copy(rows_a, acc.at[sidx_a], sem_sa, add=True)

            @pl.when(e + 1 < vnblk)
            def _():
                pltpu.make_async_copy(src(gidx_b2), rows_b, sem_b).wait()
                _scale(e + 1, rows_b)

            @pl.when(e + 2 < vnblk)
            def _():
                pltpu.make_async_copy(rows_a, acc.at[sidx_a],
                                      sem_sa).wait()
                _build(e + 2, gidx_a, sidx_a)
                pltpu.async_copy(src(gidx_a), rows_a, sem_a)

            @pl.when(e + 1 < vnblk)
            def _():
                pltpu.async_copy(rows_b, acc.at[sidx_b2], sem_sb,
                                 add=True)
            return carry
        lax.fori_loop(0, NBLK // 2, _pair, 0)

        @pl.when(vnblk > 0)
        def _():
            pltpu.make_async_copy(rows_a, acc.at[sidx_a], sem_sa).wait()

        @pl.when(vnblk > NBLK - 1)
        def _():
            pltpu.make_async_copy(rows_b, acc.at[sidx_b2], sem_sb).wait()
        plsc.subcore_barrier()

        pltpu.sync_copy(
            acc.at[pl.ds(s * ROWS_PER_TILE, ROWS_PER_TILE)],
            out_hbm.at[pl.ds(c * ROWS_PER_SC + s * ROWS_PER_TILE,
                             ROWS_PER_TILE),
                       pl.ds(p * HALF, HALF)],
        )
        plsc.subcore_barrier()


def kernel(param_enc, h_prot, W1, b1, protein_idx, indices_prot,
           global_lig_idx):
    pidx = protein_idx.astype(jnp.int32).reshape(B, 1)
    ip = indices_prot.astype(jnp.int32)
    gl = global_lig_idx.astype(jnp.int32)
    wt = W1.reshape(1, F)
    b1f = b1.astype(jnp.float32)

    gate, prow = _gate_call(param_enc, wt, b1f, pidx, ip, gl)

    hprot2 = h_prot.reshape(N_PROT * L_PROT, D)
    out = _sc_scatter(hprot2, gl.reshape(BI), prow.reshape(BI),
                      gate.reshape(BI))
    return out


# speedup vs baseline: 1.8952x; 1.8952x over previous
import functools

import jax
import jax.numpy as jnp
from jax import lax
from jax.experimental import pallas as pl
from jax.experimental.pallas import tpu as pltpu
from jax.experimental.pallas import tpu_sc as plsc

B = 256
I = 128
F = 256
N_PROT = 256
L_PROT = 512
D = 256
M = 16384
INC_PAD = -510

BI = B * I
NC = 2
NS = 16
CHUNK = BI // NS
BLK = 128
NBLK = CHUNK // BLK
HALF = D // 2
ROWS_PER_SC = M // NC
ROWS_PER_TILE = ROWS_PER_SC // NS
CPAD = CHUNK + 2 * BLK
GB = 32


def _gate_body(b1_ref, pe_ref, w_ref, pidx_ref, ip_ref, gl_ref,
               gate_ref, prow_ref):
    x = pe_ref[...]
    w = w_ref[0]
    g = jnp.sum(x * w, axis=-1) + b1_ref[0]
    g = jnp.maximum(g, 0.0)
    m = (gl_ref[...] != INC_PAD).astype(jnp.float32)
    gate_ref[...] = g * m
    prow_ref[...] = pidx_ref[...] * L_PROT + ip_ref[...]


def _gate_call(pe, wt, b1, pidx, ip, gl):
    return pl.pallas_call(
        _gate_body,
        grid=(B // GB,),
        in_specs=[
            pl.BlockSpec(memory_space=pltpu.SMEM),
            pl.BlockSpec((GB, I, F), lambda i: (i, 0, 0)),
            pl.BlockSpec((1, F), lambda i: (0, 0)),
            pl.BlockSpec((GB, 1), lambda i: (i, 0)),
            pl.BlockSpec((GB, I), lambda i: (i, 0)),
            pl.BlockSpec((GB, I), lambda i: (i, 0)),
        ],
        out_specs=[
            pl.BlockSpec((GB, I), lambda i: (i, 0)),
            pl.BlockSpec((GB, I), lambda i: (i, 0)),
        ],
        out_shape=[
            jax.ShapeDtypeStruct((B, I), jnp.float32),
            jax.ShapeDtypeStruct((B, I), jnp.int32),
        ],
    )(b1, pe, wt, pidx, ip, gl)


_mesh = plsc.VectorSubcoreMesh(core_axis_name="c", subcore_axis_name="s")


@functools.partial(
    pl.kernel,
    out_type=jax.ShapeDtypeStruct((M, D), jnp.float32),
    mesh=_mesh,
    scratch_types=[
        pltpu.VMEM_SHARED((ROWS_PER_SC, HALF), jnp.float32),
        pltpu.VMEM((CHUNK,), jnp.int32),
        pltpu.VMEM((CHUNK,), jnp.int32),
        pltpu.VMEM((CHUNK,), jnp.float32),
        pltpu.VMEM((CHUNK,), jnp.int32),
        pltpu.VMEM((CHUNK,), jnp.float32),
        pltpu.VMEM((BLK,), jnp.int32),
        pltpu.VMEM((BLK,), jnp.int32),
        pltpu.VMEM((BLK,), jnp.int32),
        pltpu.VMEM((BLK,), jnp.int32),
        pltpu.VMEM((BLK, HALF), jnp.float32),
        pltpu.VMEM((BLK, HALF), jnp.float32),
        pltpu.VMEM((BLK, HALF), jnp.float32),
        pltpu.SemaphoreType.DMA,
        pltpu.SemaphoreType.DMA,
        pltpu.SemaphoreType.DMA,
        pltpu.SemaphoreType.DMA,
    ],
    compiler_params=pltpu.CompilerParams(needs_layout_passes=False),
)
def _sc_scatter(hprot_hbm, glig_hbm, prow_hbm, gate_hbm, out_hbm,
                acc, glig_v, prow_v, gate_v, sidx_f, mgate_v,
                gidx_a, sidx_a, gidx_b2, sidx_b2,
                rows_a, rows_b, zeros_v, sem_a, sem_b, sem_sa, sem_sb):
    c = lax.axis_index("c")
    s = lax.axis_index("s")
    lo = c * ROWS_PER_SC
    base = s * CHUNK

    pltpu.sync_copy(glig_hbm.at[pl.ds(base, CHUNK)], glig_v)
    pltpu.sync_copy(prow_hbm.at[pl.ds(base, CHUNK)], prow_v)
    pltpu.sync_copy(gate_hbm.at[pl.ds(base, CHUNK)], gate_v)

    zero16 = jnp.zeros((16,), jnp.float32)
    lane = [jnp.full((16,), rr, jnp.int32) for rr in range(16)]

    def _zrow(r, carry):
        for k in range(HALF // 16):
            zeros_v[r, pl.ds(k * 16, 16)] = zero16
        return carry
    lax.fori_loop(0, BLK, _zrow, 0)

    def _prep(g, carry):
        o = g * 16
        glv = glig_v[pl.ds(o, 16)]
        owned = jnp.logical_and(glv >= lo, glv < lo + ROWS_PER_SC)
        sidx_f[pl.ds(o, 16)] = jnp.where(owned, glv - lo, 0)
        mgate_v[pl.ds(o, 16)] = jnp.where(owned, gate_v[pl.ds(o, 16)], 0.0)
        return carry
    lax.fori_loop(0, CHUNK // 16, _prep, 0)

    def _build(jb, gidx_x, sidx_x):
        b0 = jb * BLK

        def _g16(k, c2):
            o = k * 16
            gidx_x[pl.ds(o, 16)] = prow_v[pl.ds(b0 + o, 16)]
            sidx_x[pl.ds(o, 16)] = sidx_f[pl.ds(b0 + o, 16)]
            return c2
        lax.fori_loop(0, BLK // 16, _g16, 0)

    def _scale(jb, rows_x):
        b0 = jb * BLK

        def _row(t, c2):
            mg16 = mgate_v[pl.ds(b0 + t * 16, 16)]
            for rr in range(16):
                gv = mg16.at[lane[rr]].get(mode="promise_in_bounds")
                r = t * 16 + rr
                for k in range(HALF // 16):
                    sl = pl.ds(k * 16, 16)
                    rows_x[r, sl] = rows_x[r, sl] * gv
            return c2
        lax.fori_loop(0, BLK // 16, _row, 0)

    for p in range(2):
        src = lambda gidx_x: hprot_hbm.at[gidx_x, pl.ds(p * HALF, HALF)]

        for q in range(ROWS_PER_TILE // BLK):
            pltpu.sync_copy(
                zeros_v, acc.at[pl.ds(s * ROWS_PER_TILE + q * BLK, BLK)])
        plsc.subcore_barrier()

        _build(0, gidx_a, sidx_a)
        pltpu.async_copy(src(gidx_a), rows_a, sem_a)

        def _pair(j, carry):
            e = j * 2

            @pl.when(j > 0)
            def _():
                pltpu.make_async_copy(rows_b, acc.at[sidx_b2],
                                      sem_sb).wait()

            _build(e + 1, gidx_b2, sidx_b2)
            pltpu.async_copy(src(gidx_b2), rows_b, sem_b)

            pltpu.make_async_copy(src(gidx_a), rows_a, sem_a).wait()
            _scale(e, rows_a)
            pltpu.async_copy(rows_a, acc.at[sidx_a], sem_sa, add=True)

            pltpu.make_async_copy(src(gidx_b2), rows_b, sem_b).wait()
            _scale(e + 1, rows_b)

            @pl.when(j < NBLK // 2 - 1)
            def _():
                pltpu.make_async_copy(rows_a, acc.at[sidx_a],
                                      sem_sa).wait()
                _build(e + 2, gidx_a, sidx_a)
                pltpu.async_copy(src(gidx_a), rows_a, sem_a)

            pltpu.async_copy(rows_b, acc.at[sidx_b2], sem_sb, add=True)
            return carry
        lax.fori_loop(0, NBLK // 2, _pair, 0)

        pltpu.make_async_copy(rows_a, acc.at[sidx_a], sem_sa).wait()
        pltpu.make_async_copy(rows_b, acc.at[sidx_b2], sem_sb).wait()
        plsc.subcore_barrier()

        pltpu.sync_copy(
            acc.at[pl.ds(s * ROWS_PER_TILE, ROWS_PER_TILE)],
            out_hbm.at[pl.ds(c * ROWS_PER_SC + s * ROWS_PER_TILE,
                             ROWS_PER_TILE),
                       pl.ds(p * HALF, HALF)],
        )
        plsc.subcore_barrier()


def kernel(param_enc, h_prot, W1, b1, protein_idx, indices_prot,
           global_lig_idx):
    pidx = protein_idx.astype(jnp.int32).reshape(B, 1)
    ip = indices_prot.astype(jnp.int32)
    gl = global_lig_idx.astype(jnp.int32)
    wt = W1.reshape(1, F)
    b1f = b1.astype(jnp.float32)

    gate, prow = _gate_call(param_enc, wt, b1f, pidx, ip, gl)

    hprot2 = h_prot.reshape(N_PROT * L_PROT, D)
    out = _sc_scatter(hprot2, gl.reshape(BI), prow.reshape(BI),
                      gate.reshape(BI))
    return out


# speedup vs baseline: 1.9029x; 1.0041x over previous
import functools

import jax
import jax.numpy as jnp
from jax import lax
from jax.experimental import pallas as pl
from jax.experimental.pallas import tpu as pltpu
from jax.experimental.pallas import tpu_sc as plsc

B = 256
I = 128
F = 256
N_PROT = 256
L_PROT = 512
D = 256
M = 16384
INC_PAD = -510

BI = B * I
NC = 2
NS = 16
CHUNK = BI // NS
BLK = 128
NBLK = CHUNK // BLK
HALF = D // 2
ROWS_PER_SC = M // NC
ROWS_PER_TILE = ROWS_PER_SC // NS
CPAD = CHUNK + 2 * BLK
GB = 32


def _gate_body(b1_ref, pe_ref, wc_ref, pidx_ref, ip_ref, gl_ref,
               gate_ref, prow_ref):
    x = pe_ref[...].reshape(GB * I, F)
    g2 = jnp.dot(x, wc_ref[...],
                 preferred_element_type=jnp.float32)
    g = g2.reshape(GB, I) + b1_ref[0]
    g = jnp.maximum(g, 0.0)
    m = (gl_ref[...] != INC_PAD).astype(jnp.float32)
    gate_ref[...] = g * m
    prow_ref[...] = pidx_ref[...] * L_PROT + ip_ref[...]


def _gate_call(pe, wt, b1, pidx, ip, gl):
    return pl.pallas_call(
        _gate_body,
        grid=(B // GB,),
        in_specs=[
            pl.BlockSpec(memory_space=pltpu.SMEM),
            pl.BlockSpec((GB, I, F), lambda i: (i, 0, 0)),
            pl.BlockSpec((F, 1), lambda i: (0, 0)),
            pl.BlockSpec((GB, 1), lambda i: (i, 0)),
            pl.BlockSpec((GB, I), lambda i: (i, 0)),
            pl.BlockSpec((GB, I), lambda i: (i, 0)),
        ],
        out_specs=[
            pl.BlockSpec((GB, I), lambda i: (i, 0)),
            pl.BlockSpec((GB, I), lambda i: (i, 0)),
        ],
        out_shape=[
            jax.ShapeDtypeStruct((B, I), jnp.float32),
            jax.ShapeDtypeStruct((B, I), jnp.int32),
        ],
    )(b1, pe, wt, pidx, ip, gl)


_mesh = plsc.VectorSubcoreMesh(core_axis_name="c", subcore_axis_name="s")


@functools.partial(
    pl.kernel,
    out_type=jax.ShapeDtypeStruct((M, D), jnp.float32),
    mesh=_mesh,
    scratch_types=[
        pltpu.VMEM_SHARED((ROWS_PER_SC, HALF), jnp.float32),
        pltpu.VMEM((CHUNK,), jnp.int32),
        pltpu.VMEM((CHUNK,), jnp.int32),
        pltpu.VMEM((CHUNK,), jnp.float32),
        pltpu.VMEM((CHUNK,), jnp.int32),
        pltpu.VMEM((CHUNK,), jnp.float32),
        pltpu.VMEM((BLK,), jnp.int32),
        pltpu.VMEM((BLK,), jnp.int32),
        pltpu.VMEM((BLK,), jnp.int32),
        pltpu.VMEM((BLK,), jnp.int32),
        pltpu.VMEM((BLK, HALF), jnp.float32),
        pltpu.VMEM((BLK, HALF), jnp.float32),
        pltpu.VMEM((BLK, HALF), jnp.float32),
        pltpu.SemaphoreType.DMA,
        pltpu.SemaphoreType.DMA,
        pltpu.SemaphoreType.DMA,
        pltpu.SemaphoreType.DMA,
    ],
    compiler_params=pltpu.CompilerParams(needs_layout_passes=False),
)
def _sc_scatter(hprot_hbm, glig_hbm, prow_hbm, gate_hbm, out_hbm,
                acc, glig_v, prow_v, gate_v, sidx_f, mgate_v,
                gidx_a, sidx_a, gidx_b2, sidx_b2,
                rows_a, rows_b, zeros_v, sem_a, sem_b, sem_sa, sem_sb):
    c = lax.axis_index("c")
    s = lax.axis_index("s")
    lo = c * ROWS_PER_SC
    base = s * CHUNK

    pltpu.sync_copy(glig_hbm.at[pl.ds(base, CHUNK)], glig_v)
    pltpu.sync_copy(prow_hbm.at[pl.ds(base, CHUNK)], prow_v)
    pltpu.sync_copy(gate_hbm.at[pl.ds(base, CHUNK)], gate_v)

    zero16 = jnp.zeros((16,), jnp.float32)
    lane = [jnp.full((16,), rr, jnp.int32) for rr in range(16)]

    def _zrow(r, carry):
        for k in range(HALF // 16):
            zeros_v[r, pl.ds(k * 16, 16)] = zero16
        return carry
    lax.fori_loop(0, BLK, _zrow, 0)

    def _prep(g, carry):
        o = g * 16
        glv = glig_v[pl.ds(o, 16)]
        owned = jnp.logical_and(glv >= lo, glv < lo + ROWS_PER_SC)
        sidx_f[pl.ds(o, 16)] = jnp.where(owned, glv - lo, 0)
        mgate_v[pl.ds(o, 16)] = jnp.where(owned, gate_v[pl.ds(o, 16)], 0.0)
        return carry
    lax.fori_loop(0, CHUNK // 16, _prep, 0)

    def _build(jb, gidx_x, sidx_x):
        b0 = jb * BLK

        def _g16(k, c2):
            o = k * 16
            gidx_x[pl.ds(o, 16)] = prow_v[pl.ds(b0 + o, 16)]
            sidx_x[pl.ds(o, 16)] = sidx_f[pl.ds(b0 + o, 16)]
            return c2
        lax.fori_loop(0, BLK // 16, _g16, 0)

    def _scale(jb, rows_x):
        b0 = jb * BLK

        def _row(t, c2):
            mg16 = mgate_v[pl.ds(b0 + t * 16, 16)]
            for rr in range(16):
                gv = mg16.at[lane[rr]].get(mode="promise_in_bounds")
                r = t * 16 + rr
                for k in range(HALF // 16):
                    sl = pl.ds(k * 16, 16)
                    rows_x[r, sl] = rows_x[r, sl] * gv
            return c2
        lax.fori_loop(0, BLK // 16, _row, 0)

    for p in range(2):
        src = lambda gidx_x: hprot_hbm.at[gidx_x, pl.ds(p * HALF, HALF)]

        for q in range(ROWS_PER_TILE // BLK):
            pltpu.sync_copy(
                zeros_v, acc.at[pl.ds(s * ROWS_PER_TILE + q * BLK, BLK)])
        plsc.subcore_barrier()

        _build(0, gidx_a, sidx_a)
        pltpu.async_copy(src(gidx_a), rows_a, sem_a)

        def _pair(j, carry):
            e = j * 2

            @pl.when(j > 0)
            def _():
                pltpu.make_async_copy(rows_b, acc.at[sidx_b2],
                                      sem_sb).wait()

            _build(e + 1, gidx_b2, sidx_b2)
            pltpu.async_copy(src(gidx_b2), rows_b, sem_b)

            pltpu.make_async_copy(src(gidx_a), rows_a, sem_a).wait()
            _scale(e, rows_a)
            pltpu.async_copy(rows_a, acc.at[sidx_a], sem_sa, add=True)

            pltpu.make_async_copy(src(gidx_b2), rows_b, sem_b).wait()
            _scale(e + 1, rows_b)

            @pl.when(j < NBLK // 2 - 1)
            def _():
                pltpu.make_async_copy(rows_a, acc.at[sidx_a],
                                      sem_sa).wait()
                _build(e + 2, gidx_a, sidx_a)
                pltpu.async_copy(src(gidx_a), rows_a, sem_a)

            pltpu.async_copy(rows_b, acc.at[sidx_b2], sem_sb, add=True)
            return carry
        lax.fori_loop(0, NBLK // 2, _pair, 0)

        pltpu.make_async_copy(rows_a, acc.at[sidx_a], sem_sa).wait()
        pltpu.make_async_copy(rows_b, acc.at[sidx_b2], sem_sb).wait()
        plsc.subcore_barrier()

        pltpu.sync_copy(
            acc.at[pl.ds(s * ROWS_PER_TILE, ROWS_PER_TILE)],
            out_hbm.at[pl.ds(c * ROWS_PER_SC + s * ROWS_PER_TILE,
                             ROWS_PER_TILE),
                       pl.ds(p * HALF, HALF)],
        )
        plsc.subcore_barrier()


def kernel(param_enc, h_prot, W1, b1, protein_idx, indices_prot,
           global_lig_idx):
    pidx = protein_idx.astype(jnp.int32).reshape(B, 1)
    ip = indices_prot.astype(jnp.int32)
    gl = global_lig_idx.astype(jnp.int32)
    wc = W1.reshape(F, 1)
    b1f = b1.astype(jnp.float32)

    gate, prow = _gate_call(param_enc, wc, b1f, pidx, ip, gl)

    hprot2 = h_prot.reshape(N_PROT * L_PROT, D)
    out = _sc_scatter(hprot2, gl.reshape(BI), prow.reshape(BI),
                      gate.reshape(BI))
    return out
